# baseline (speedup 1.0000x reference)
import functools

import jax
import jax.numpy as jnp
from jax import lax
from jax.experimental import pallas as pl
from jax.experimental.pallas import tpu as pltpu
from jax.experimental.pallas import tpu_sc as plsc

E = 8
K = 2
D = 1024
F = 2048
T = 2048

BM = 128
NPAD = T * K + E * BM
NB = NPAD // BM
NW = 32
RPW = NPAD // NW
CH = 32
TPW = T // NW
C2 = 16
NSLOT = 2


def _routing_body(x_ref, gw_ref, pos1_ref, pos2_ref, g1_ref, g2_ref, be_ref):
    x = x_ref[...]
    logits = jnp.dot(x, gw_ref[...], preferred_element_type=jnp.float32)
    idx = lax.broadcasted_iota(jnp.int32, (T, E), 1)
    m1 = jnp.max(logits, axis=1, keepdims=True)
    i1 = jnp.min(jnp.where(logits == m1, idx, E), axis=1, keepdims=True)
    masked = jnp.where(idx == i1, -jnp.inf, logits)
    m2 = jnp.max(masked, axis=1, keepdims=True)
    i2 = jnp.min(jnp.where(masked == m2, idx, E), axis=1, keepdims=True)
    g1_ref[...] = jax.nn.sigmoid(m1 - m2).reshape(T // 128, 128)
    g2_ref[...] = jax.nn.sigmoid(m2 - m1).reshape(T // 128, 128)

    ind = jnp.where(idx == i1, 1.0, 0.0) + jnp.where(idx == i2, 1.0, 0.0)
    cum = ind
    s = 1
    while s < T:
        cum = cum + jnp.concatenate(
            [jnp.zeros((s, E), jnp.float32), cum[: T - s]], axis=0)
        s *= 2
    rank = cum - ind
    counts = cum[T - 1:T, :]
    pc = (((counts.astype(jnp.int32) + BM - 1) // BM) * BM).astype(jnp.float32)
    r8 = lax.broadcasted_iota(jnp.int32, (E, E), 0)
    c8 = lax.broadcasted_iota(jnp.int32, (E, E), 1)
    utri8 = jnp.where(r8 < c8, 1.0, 0.0)
    po = jnp.dot(pc, utri8, preferred_element_type=jnp.float32)
    posmat = po + rank
    pos1_ref[...] = jnp.sum(jnp.where(idx == i1, posmat, 0.0),
                            axis=1, keepdims=True).astype(jnp.int32).reshape(
                                T // 128, 128)
    pos2_ref[...] = jnp.sum(jnp.where(idx == i2, posmat, 0.0),
                            axis=1, keepdims=True).astype(jnp.int32).reshape(
                                T // 128, 128)
    po_end = po + pc
    b128 = lax.broadcasted_iota(jnp.int32, (128, E), 0).astype(jnp.float32) * BM
    ucnt = jnp.sum(jnp.where(b128 >= po_end, 1, 0), axis=1, keepdims=True)
    bec = jnp.minimum(ucnt, E - 1)
    prev = jnp.concatenate([jnp.full((1, 1), -1, jnp.int32), bec[:127]], axis=0)
    chg = (bec != prev).astype(jnp.int32)
    run = chg
    s = 1
    while s < 128:
        run = run + jnp.concatenate(
            [jnp.zeros((s, 1), jnp.int32), run[: 128 - s]], axis=0)
        s *= 2
    slot = (run - 1) % 2
    e8 = lax.broadcasted_iota(jnp.int32, (128, E), 1)
    pres = pc > 0.0
    cand = jnp.where((e8 > bec) & pres, e8, E)
    nxt0 = jnp.min(cand, axis=1, keepdims=True)
    nxt = jnp.where(nxt0 >= E, bec, nxt0)
    act = (ucnt < E).astype(jnp.int32)
    zero = jnp.zeros((128, 1), jnp.int32)
    be_ref[...] = jnp.concatenate(
        [bec, chg, slot, nxt, act, zero, zero, zero], axis=1)


def _routing(inputs, gate_w):
    return pl.pallas_call(
        _routing_body,
        out_shape=(
            jax.ShapeDtypeStruct((T // 128, 128), jnp.int32),
            jax.ShapeDtypeStruct((T // 128, 128), jnp.int32),
            jax.ShapeDtypeStruct((T // 128, 128), jnp.float32),
            jax.ShapeDtypeStruct((T // 128, 128), jnp.float32),
            jax.ShapeDtypeStruct((128, E), jnp.int32),
        ),
    )(inputs, gate_w)


def _dispatch_body(pos1_hbm, pos2_hbm, x_hbm, xd_hbm,
                   p1v, p2v, rowbuf, sem):
    wid = lax.axis_index("s") * 2 + lax.axis_index("c")
    tb = wid * TPW
    pltpu.sync_copy(pos1_hbm.at[pl.ds(tb, TPW)], p1v)
    pltpu.sync_copy(pos2_hbm.at[pl.ds(tb, TPW)], p2v)
    pltpu.async_copy(x_hbm.at[pl.ds(tb, TPW)], rowbuf, sem).wait()
    pltpu.sync_copy(rowbuf, xd_hbm.at[p1v])
    pltpu.sync_copy(rowbuf, xd_hbm.at[p2v])


def _dispatch(pos1f, pos2f, inputs):
    fn = functools.partial(
        pl.kernel,
        out_type=jax.ShapeDtypeStruct((NPAD, D), jnp.float32),
        mesh=plsc.VectorSubcoreMesh(core_axis_name="c", subcore_axis_name="s"),
        compiler_params=pltpu.CompilerParams(needs_layout_passes=False),
        scratch_types=[
            pltpu.VMEM((TPW,), jnp.int32),
            pltpu.VMEM((TPW,), jnp.int32),
            pltpu.VMEM((TPW, D), jnp.float32),
            pltpu.SemaphoreType.DMA,
        ],
    )(_dispatch_body)
    return fn(pos1f, pos2f, inputs)


def _ffn_body(s_ref, x_ref, w1_hbm, w2_hbm, o_ref, w1b, w2b, sem0, sem1):
    b = pl.program_id(0)
    e = s_ref[b, 0]
    chg = s_ref[b, 1]
    slot = s_ref[b, 2]
    nxt = s_ref[b, 3]

    def start(dst, eidx, sem):
        pltpu.make_async_copy(w1_hbm.at[eidx], w1b.at[dst], sem).start()
        pltpu.make_async_copy(w2_hbm.at[eidx], w2b.at[dst], sem).start()

    def drain(dst, sem):
        pltpu.make_async_copy(w1_hbm.at[0], w1b.at[dst], sem).wait()
        pltpu.make_async_copy(w2_hbm.at[0], w2b.at[dst], sem).wait()

    @pl.when(b == 0)
    def _():
        start(0, e, sem0)

        @pl.when(nxt != e)
        def _():
            start(1, nxt, sem1)
        drain(0, sem0)

    @pl.when((b > 0) & (chg == 1) & (slot == 0))
    def _():
        drain(0, sem0)

        @pl.when(nxt != e)
        def _():
            start(1, nxt, sem1)

    @pl.when((b > 0) & (chg == 1) & (slot == 1))
    def _():
        drain(1, sem1)

        @pl.when(nxt != e)
        def _():
            start(0, nxt, sem0)

    x = x_ref[...]
    h = jnp.dot(x, w1b[slot], preferred_element_type=jnp.float32)
    h = h * jax.nn.sigmoid(h)
    o_ref[...] = jnp.dot(h, w2b[slot], preferred_element_type=jnp.float32)


def _ffn(meta, xd, w1, w2):
    grid_spec = pltpu.PrefetchScalarGridSpec(
        num_scalar_prefetch=1,
        grid=(NB,),
        in_specs=[
            pl.BlockSpec((BM, D), lambda b, s: (b, 0)),
            pl.BlockSpec(memory_space=pl.ANY),
            pl.BlockSpec(memory_space=pl.ANY),
        ],
        out_specs=pl.BlockSpec((BM, D), lambda b, s: (b, 0)),
        scratch_shapes=[
            pltpu.VMEM((2, D, F), jnp.float32),
            pltpu.VMEM((2, F, D), jnp.float32),
            pltpu.SemaphoreType.DMA,
            pltpu.SemaphoreType.DMA,
        ],
    )
    return pl.pallas_call(
        _ffn_body,
        grid_spec=grid_spec,
        out_shape=jax.ShapeDtypeStruct((NPAD, D), jnp.float32),
    )(meta, xd, w1, w2)


def _combine_body(y_hbm, pos1_hbm, pos2_hbm, g1_hbm, g2_hbm, out_hbm,
                  p1v, p2v, g1v, g2v, bufa, bufb, gsem, wsem):
    wid = lax.axis_index("s") * 2 + lax.axis_index("c")
    tb = wid * TPW
    pltpu.sync_copy(pos1_hbm.at[pl.ds(tb, TPW)], p1v)
    pltpu.sync_copy(pos2_hbm.at[pl.ds(tb, TPW)], p2v)
    pltpu.sync_copy(g1_hbm.at[pl.ds(tb, TPW)], g1v)
    pltpu.sync_copy(g2_hbm.at[pl.ds(tb, TPW)], g2v)
    nch = TPW // C2

    def issue(c):
        off = c * C2
        slot = c % NSLOT
        d1 = pltpu.async_copy(y_hbm.at[p1v.at[pl.ds(off, C2)]],
                              bufa.at[slot], gsem)
        d2 = pltpu.async_copy(y_hbm.at[p2v.at[pl.ds(off, C2)]],
                              bufb.at[slot], gsem)
        return d1, d2

    gds = [None] * nch
    wds = [None] * nch
    for c in range(NSLOT):
        gds[c] = issue(c)
    for c in range(nch):
        slot = c % NSLOT
        off = c * C2
        gds[c][0].wait()
        gds[c][1].wait()

        def rbody(r, _):
            i16 = jnp.broadcast_to(off + r, (16,))
            ga = plsc.load_gather(g1v, [i16])
            gb = plsc.load_gather(g2v, [i16])
            for j in range(D // 16):
                sl = pl.ds(j * 16, 16)
                bufa[slot, r, sl] = ga * bufa[slot, r, sl] + gb * bufb[slot, r, sl]
            return 0
        lax.fori_loop(0, C2, rbody, 0)
        wds[c] = pltpu.async_copy(bufa.at[slot],
                                  out_hbm.at[pl.ds(tb + off, C2)], wsem)
        if c + NSLOT < nch:
            wds[c].wait()
            gds[c + NSLOT] = issue(c + NSLOT)
    for c in range(max(0, nch - NSLOT), nch):
        wds[c].wait()


def _combine(y, pos1f, pos2f, g1f, g2f):
    fn = functools.partial(
        pl.kernel,
        out_type=jax.ShapeDtypeStruct((T, D), jnp.float32),
        mesh=plsc.VectorSubcoreMesh(core_axis_name="c", subcore_axis_name="s"),
        compiler_params=pltpu.CompilerParams(needs_layout_passes=False),
        scratch_types=[
            pltpu.VMEM((TPW,), jnp.int32),
            pltpu.VMEM((TPW,), jnp.int32),
            pltpu.VMEM((TPW,), jnp.float32),
            pltpu.VMEM((TPW,), jnp.float32),
            pltpu.VMEM((NSLOT, C2, D), jnp.float32),
            pltpu.VMEM((NSLOT, C2, D), jnp.float32),
            pltpu.SemaphoreType.DMA,
            pltpu.SemaphoreType.DMA,
        ],
    )(_combine_body)
    return fn(y, pos1f, pos2f, g1f, g2f)


@jax.jit
def kernel(inputs, gate_w, w1, w2):
    pos1, pos2, g1, g2, be = _routing(inputs, gate_w)
    pos1f = pos1.reshape(T)
    pos2f = pos2.reshape(T)
    xd = _dispatch(pos1f, pos2f, inputs)
    y = _ffn(be, xd, w1, w2)
    return _combine(y, pos1f, pos2f, g1.reshape(T), g2.reshape(T))


# speedup vs baseline: 1.0095x; 1.0095x over previous
import functools

import jax
import jax.numpy as jnp
from jax import lax
from jax.experimental import pallas as pl
from jax.experimental.pallas import tpu as pltpu
from jax.experimental.pallas import tpu_sc as plsc

E = 8
K = 2
D = 1024
F = 2048
T = 2048

BM = 128
NPAD = T * K + E * BM
NB = NPAD // BM
NW = 32
RPW = NPAD // NW
CH = 32
TPW = T // NW
C2 = 16
NSLOT = 2


def _routing_body(x_ref, gw_ref, pos1_ref, pos2_ref, g1_ref, g2_ref, be_ref):
    x = x_ref[...]
    logits = jnp.dot(x, gw_ref[...], preferred_element_type=jnp.float32)
    idx = lax.broadcasted_iota(jnp.int32, (T, E), 1)
    m1 = jnp.max(logits, axis=1, keepdims=True)
    i1 = jnp.min(jnp.where(logits == m1, idx, E), axis=1, keepdims=True)
    masked = jnp.where(idx == i1, -jnp.inf, logits)
    m2 = jnp.max(masked, axis=1, keepdims=True)
    i2 = jnp.min(jnp.where(masked == m2, idx, E), axis=1, keepdims=True)
    g1_ref[...] = jax.nn.sigmoid(m1 - m2).reshape(T // 128, 128)
    g2_ref[...] = jax.nn.sigmoid(m2 - m1).reshape(T // 128, 128)

    ind = jnp.where(idx == i1, 1.0, 0.0) + jnp.where(idx == i2, 1.0, 0.0)
    cum = ind
    s = 1
    while s < T:
        cum = cum + jnp.concatenate(
            [jnp.zeros((s, E), jnp.float32), cum[: T - s]], axis=0)
        s *= 2
    rank = cum - ind
    counts = cum[T - 1:T, :]
    pc = (((counts.astype(jnp.int32) + BM - 1) // BM) * BM).astype(jnp.float32)
    r8 = lax.broadcasted_iota(jnp.int32, (E, E), 0)
    c8 = lax.broadcasted_iota(jnp.int32, (E, E), 1)
    utri8 = jnp.where(r8 < c8, 1.0, 0.0)
    po = jnp.dot(pc, utri8, preferred_element_type=jnp.float32)
    posmat = po + rank
    pos1_ref[...] = jnp.sum(jnp.where(idx == i1, posmat, 0.0),
                            axis=1, keepdims=True).astype(jnp.int32).reshape(
                                T // 128, 128)
    pos2_ref[...] = jnp.sum(jnp.where(idx == i2, posmat, 0.0),
                            axis=1, keepdims=True).astype(jnp.int32).reshape(
                                T // 128, 128)
    po_end = po + pc
    b128 = lax.broadcasted_iota(jnp.int32, (128, E), 0).astype(jnp.float32) * BM
    be_ref[...] = jnp.sum(jnp.where(b128 >= po_end, 1, 0),
                          axis=1, keepdims=True).astype(jnp.int32)


def _routing(inputs, gate_w):
    return pl.pallas_call(
        _routing_body,
        out_shape=(
            jax.ShapeDtypeStruct((T // 128, 128), jnp.int32),
            jax.ShapeDtypeStruct((T // 128, 128), jnp.int32),
            jax.ShapeDtypeStruct((T // 128, 128), jnp.float32),
            jax.ShapeDtypeStruct((T // 128, 128), jnp.float32),
            jax.ShapeDtypeStruct((128, 1), jnp.int32),
        ),
    )(inputs, gate_w)


def _dispatch_body(pos1_hbm, pos2_hbm, x_hbm, xd_hbm,
                   p1v, p2v, rowbuf, sem):
    wid = lax.axis_index("s") * 2 + lax.axis_index("c")
    tb = wid * TPW
    pltpu.sync_copy(pos1_hbm.at[pl.ds(tb, TPW)], p1v)
    pltpu.sync_copy(pos2_hbm.at[pl.ds(tb, TPW)], p2v)
    pltpu.async_copy(x_hbm.at[pl.ds(tb, TPW)], rowbuf, sem).wait()
    pltpu.sync_copy(rowbuf, xd_hbm.at[p1v])
    pltpu.sync_copy(rowbuf, xd_hbm.at[p2v])


def _dispatch(pos1f, pos2f, inputs):
    fn = functools.partial(
        pl.kernel,
        out_type=jax.ShapeDtypeStruct((NPAD, D), jnp.float32),
        mesh=plsc.VectorSubcoreMesh(core_axis_name="c", subcore_axis_name="s"),
        compiler_params=pltpu.CompilerParams(needs_layout_passes=False),
        scratch_types=[
            pltpu.VMEM((TPW,), jnp.int32),
            pltpu.VMEM((TPW,), jnp.int32),
            pltpu.VMEM((TPW, D), jnp.float32),
            pltpu.SemaphoreType.DMA,
        ],
    )(_dispatch_body)
    return fn(pos1f, pos2f, inputs)


def _ffn_body(s_ref, x_ref, w1_hbm, w2_hbm, o_ref, w1b, w2b, sem0, sem1):
    b = pl.program_id(0)
    e = s_ref[0, b]
    chg = s_ref[1, b]
    slot = s_ref[2, b]
    nxt = s_ref[3, b]

    def start(dst, eidx, sem):
        pltpu.make_async_copy(w1_hbm.at[eidx], w1b.at[dst], sem).start()
        pltpu.make_async_copy(w2_hbm.at[eidx], w2b.at[dst], sem).start()

    def drain(dst, sem):
        pltpu.make_async_copy(w1_hbm.at[0], w1b.at[dst], sem).wait()
        pltpu.make_async_copy(w2_hbm.at[0], w2b.at[dst], sem).wait()

    @pl.when(b == 0)
    def _():
        start(0, e, sem0)

        @pl.when(nxt != e)
        def _():
            start(1, nxt, sem1)
        drain(0, sem0)

    @pl.when((b > 0) & (chg == 1) & (slot == 0))
    def _():
        drain(0, sem0)

        @pl.when(nxt != e)
        def _():
            start(1, nxt, sem1)

    @pl.when((b > 0) & (chg == 1) & (slot == 1))
    def _():
        drain(1, sem1)

        @pl.when(nxt != e)
        def _():
            start(0, nxt, sem0)

    x = x_ref[...]
    h = jnp.dot(x, w1b[slot], preferred_element_type=jnp.float32)
    h = h * jax.nn.sigmoid(h)
    o_ref[...] = jnp.dot(h, w2b[slot], preferred_element_type=jnp.float32)


def _ffn(meta, xd, w1, w2):
    grid_spec = pltpu.PrefetchScalarGridSpec(
        num_scalar_prefetch=1,
        grid=(NB,),
        in_specs=[
            pl.BlockSpec((BM, D), lambda b, s: (b, 0)),
            pl.BlockSpec(memory_space=pl.ANY),
            pl.BlockSpec(memory_space=pl.ANY),
        ],
        out_specs=pl.BlockSpec((BM, D), lambda b, s: (b, 0)),
        scratch_shapes=[
            pltpu.VMEM((2, D, F), jnp.float32),
            pltpu.VMEM((2, F, D), jnp.float32),
            pltpu.SemaphoreType.DMA,
            pltpu.SemaphoreType.DMA,
        ],
    )
    return pl.pallas_call(
        _ffn_body,
        grid_spec=grid_spec,
        out_shape=jax.ShapeDtypeStruct((NPAD, D), jnp.float32),
    )(meta, xd, w1, w2)


def _combine_body(y_hbm, pos1_hbm, pos2_hbm, g1_hbm, g2_hbm, out_hbm,
                  p1v, p2v, g1v, g2v, bufa, bufb, gsem, wsem):
    wid = lax.axis_index("s") * 2 + lax.axis_index("c")
    tb = wid * TPW
    pltpu.sync_copy(pos1_hbm.at[pl.ds(tb, TPW)], p1v)
    pltpu.sync_copy(pos2_hbm.at[pl.ds(tb, TPW)], p2v)
    pltpu.sync_copy(g1_hbm.at[pl.ds(tb, TPW)], g1v)
    pltpu.sync_copy(g2_hbm.at[pl.ds(tb, TPW)], g2v)
    nch = TPW // C2

    def issue(c):
        off = c * C2
        slot = c % NSLOT
        d1 = pltpu.async_copy(y_hbm.at[p1v.at[pl.ds(off, C2)]],
                              bufa.at[slot], gsem)
        d2 = pltpu.async_copy(y_hbm.at[p2v.at[pl.ds(off, C2)]],
                              bufb.at[slot], gsem)
        return d1, d2

    gds = [None] * nch
    wds = [None] * nch
    for c in range(NSLOT):
        gds[c] = issue(c)
    for c in range(nch):
        slot = c % NSLOT
        off = c * C2
        gds[c][0].wait()
        gds[c][1].wait()

        def rbody(r, _):
            i16 = jnp.broadcast_to(off + r, (16,))
            ga = plsc.load_gather(g1v, [i16])
            gb = plsc.load_gather(g2v, [i16])
            for j in range(D // 16):
                sl = pl.ds(j * 16, 16)
                bufa[slot, r, sl] = ga * bufa[slot, r, sl] + gb * bufb[slot, r, sl]
            return 0
        lax.fori_loop(0, C2, rbody, 0)
        wds[c] = pltpu.async_copy(bufa.at[slot],
                                  out_hbm.at[pl.ds(tb + off, C2)], wsem)
        if c + NSLOT < nch:
            wds[c].wait()
            gds[c + NSLOT] = issue(c + NSLOT)
    for c in range(max(0, nch - NSLOT), nch):
        wds[c].wait()


def _combine(y, pos1f, pos2f, g1f, g2f):
    fn = functools.partial(
        pl.kernel,
        out_type=jax.ShapeDtypeStruct((T, D), jnp.float32),
        mesh=plsc.VectorSubcoreMesh(core_axis_name="c", subcore_axis_name="s"),
        compiler_params=pltpu.CompilerParams(needs_layout_passes=False),
        scratch_types=[
            pltpu.VMEM((TPW,), jnp.int32),
            pltpu.VMEM((TPW,), jnp.int32),
            pltpu.VMEM((TPW,), jnp.float32),
            pltpu.VMEM((TPW,), jnp.float32),
            pltpu.VMEM((NSLOT, C2, D), jnp.float32),
            pltpu.VMEM((NSLOT, C2, D), jnp.float32),
            pltpu.SemaphoreType.DMA,
            pltpu.SemaphoreType.DMA,
        ],
    )(_combine_body)
    return fn(y, pos1f, pos2f, g1f, g2f)


@jax.jit
def kernel(inputs, gate_w, w1, w2):
    pos1, pos2, g1, g2, be = _routing(inputs, gate_w)
    pos1f = pos1.reshape(T)
    pos2f = pos2.reshape(T)
    xd = _dispatch(pos1f, pos2f, inputs)
    ucnt = be.reshape(128)[:NB]
    be_s = jnp.minimum(ucnt, E - 1)
    chg = jnp.concatenate(
        [jnp.ones((1,), jnp.int32), (be_s[1:] != be_s[:-1]).astype(jnp.int32)])
    run_id = jnp.cumsum(chg) - 1
    slot = run_id % 2
    re = jnp.zeros((NB,), jnp.int32).at[run_id].set(be_s)
    nxt = re[jnp.minimum(run_id + 1, run_id[-1])]
    meta = jnp.stack([be_s, chg, slot, nxt], axis=0)
    y = _ffn(meta, xd, w1, w2)
    return _combine(y, pos1f, pos2f, g1.reshape(T), g2.reshape(T))


# speedup vs baseline: 1.0168x; 1.0073x over previous
import functools

import jax
import jax.numpy as jnp
from jax import lax
from jax.experimental import pallas as pl
from jax.experimental.pallas import tpu as pltpu
from jax.experimental.pallas import tpu_sc as plsc

E = 8
K = 2
D = 1024
F = 2048
T = 2048

BM = 128
NPAD = T * K + E * BM
NB = NPAD // BM
NW = 32
RPW = NPAD // NW
CH = 32
TPW = T // NW
C2 = 16
NSLOT = 3


def _routing_body(x_ref, gw_ref, pos1_ref, pos2_ref, g1_ref, g2_ref, be_ref):
    x = x_ref[...]
    logits = jnp.dot(x, gw_ref[...], preferred_element_type=jnp.float32)
    idx = lax.broadcasted_iota(jnp.int32, (T, E), 1)
    m1 = jnp.max(logits, axis=1, keepdims=True)
    i1 = jnp.min(jnp.where(logits == m1, idx, E), axis=1, keepdims=True)
    masked = jnp.where(idx == i1, -jnp.inf, logits)
    m2 = jnp.max(masked, axis=1, keepdims=True)
    i2 = jnp.min(jnp.where(masked == m2, idx, E), axis=1, keepdims=True)
    g1_ref[...] = jax.nn.sigmoid(m1 - m2).reshape(T // 128, 128)
    g2_ref[...] = jax.nn.sigmoid(m2 - m1).reshape(T // 128, 128)

    ind = jnp.where(idx == i1, 1.0, 0.0) + jnp.where(idx == i2, 1.0, 0.0)
    cum = ind
    s = 1
    while s < T:
        cum = cum + jnp.concatenate(
            [jnp.zeros((s, E), jnp.float32), cum[: T - s]], axis=0)
        s *= 2
    rank = cum - ind
    counts = cum[T - 1:T, :]
    pc = (((counts.astype(jnp.int32) + BM - 1) // BM) * BM).astype(jnp.float32)
    r8 = lax.broadcasted_iota(jnp.int32, (E, E), 0)
    c8 = lax.broadcasted_iota(jnp.int32, (E, E), 1)
    utri8 = jnp.where(r8 < c8, 1.0, 0.0)
    po = jnp.dot(pc, utri8, preferred_element_type=jnp.float32)
    posmat = po + rank
    pos1_ref[...] = jnp.sum(jnp.where(idx == i1, posmat, 0.0),
                            axis=1, keepdims=True).astype(jnp.int32).reshape(
                                T // 128, 128)
    pos2_ref[...] = jnp.sum(jnp.where(idx == i2, posmat, 0.0),
                            axis=1, keepdims=True).astype(jnp.int32).reshape(
                                T // 128, 128)
    po_end = po + pc
    b128 = lax.broadcasted_iota(jnp.int32, (128, E), 0).astype(jnp.float32) * BM
    be_ref[...] = jnp.sum(jnp.where(b128 >= po_end, 1, 0),
                          axis=1, keepdims=True).astype(jnp.int32)


def _routing(inputs, gate_w):
    return pl.pallas_call(
        _routing_body,
        out_shape=(
            jax.ShapeDtypeStruct((T // 128, 128), jnp.int32),
            jax.ShapeDtypeStruct((T // 128, 128), jnp.int32),
            jax.ShapeDtypeStruct((T // 128, 128), jnp.float32),
            jax.ShapeDtypeStruct((T // 128, 128), jnp.float32),
            jax.ShapeDtypeStruct((128, 1), jnp.int32),
        ),
    )(inputs, gate_w)


def _dispatch_body(pos1_hbm, pos2_hbm, x_hbm, xd_hbm,
                   p1v, p2v, rowbuf, sem):
    wid = lax.axis_index("s") * 2 + lax.axis_index("c")
    tb = wid * TPW
    pltpu.sync_copy(pos1_hbm.at[pl.ds(tb, TPW)], p1v)
    pltpu.sync_copy(pos2_hbm.at[pl.ds(tb, TPW)], p2v)
    pltpu.async_copy(x_hbm.at[pl.ds(tb, TPW)], rowbuf, sem).wait()
    pltpu.sync_copy(rowbuf, xd_hbm.at[p1v])
    pltpu.sync_copy(rowbuf, xd_hbm.at[p2v])


def _dispatch(pos1f, pos2f, inputs):
    fn = functools.partial(
        pl.kernel,
        out_type=jax.ShapeDtypeStruct((NPAD, D), jnp.float32),
        mesh=plsc.VectorSubcoreMesh(core_axis_name="c", subcore_axis_name="s"),
        compiler_params=pltpu.CompilerParams(needs_layout_passes=False),
        scratch_types=[
            pltpu.VMEM((TPW,), jnp.int32),
            pltpu.VMEM((TPW,), jnp.int32),
            pltpu.VMEM((TPW, D), jnp.float32),
            pltpu.SemaphoreType.DMA,
        ],
    )(_dispatch_body)
    return fn(pos1f, pos2f, inputs)


def _ffn_body(s_ref, x_ref, w1_hbm, w2_hbm, o_ref, w1b, w2b, sem0, sem1):
    b = pl.program_id(0)
    e = s_ref[0, b]
    chg = s_ref[1, b]
    slot = s_ref[2, b]
    nxt = s_ref[3, b]

    def start(dst, eidx, sem):
        pltpu.make_async_copy(w1_hbm.at[eidx], w1b.at[dst], sem).start()
        pltpu.make_async_copy(w2_hbm.at[eidx], w2b.at[dst], sem).start()

    def drain(dst, sem):
        pltpu.make_async_copy(w1_hbm.at[0], w1b.at[dst], sem).wait()
        pltpu.make_async_copy(w2_hbm.at[0], w2b.at[dst], sem).wait()

    @pl.when(b == 0)
    def _():
        start(0, e, sem0)

        @pl.when(nxt != e)
        def _():
            start(1, nxt, sem1)
        drain(0, sem0)

    @pl.when((b > 0) & (chg == 1) & (slot == 0))
    def _():
        drain(0, sem0)

        @pl.when(nxt != e)
        def _():
            start(1, nxt, sem1)

    @pl.when((b > 0) & (chg == 1) & (slot == 1))
    def _():
        drain(1, sem1)

        @pl.when(nxt != e)
        def _():
            start(0, nxt, sem0)

    x = x_ref[...]
    h = jnp.dot(x, w1b[slot], preferred_element_type=jnp.float32)
    h = h * jax.nn.sigmoid(h)
    o_ref[...] = jnp.dot(h, w2b[slot], preferred_element_type=jnp.float32)


def _ffn(meta, xd, w1, w2):
    grid_spec = pltpu.PrefetchScalarGridSpec(
        num_scalar_prefetch=1,
        grid=(NB,),
        in_specs=[
            pl.BlockSpec((BM, D), lambda b, s: (b, 0)),
            pl.BlockSpec(memory_space=pl.ANY),
            pl.BlockSpec(memory_space=pl.ANY),
        ],
        out_specs=pl.BlockSpec((BM, D), lambda b, s: (b, 0)),
        scratch_shapes=[
            pltpu.VMEM((2, D, F), jnp.float32),
            pltpu.VMEM((2, F, D), jnp.float32),
            pltpu.SemaphoreType.DMA,
            pltpu.SemaphoreType.DMA,
        ],
    )
    return pl.pallas_call(
        _ffn_body,
        grid_spec=grid_spec,
        out_shape=jax.ShapeDtypeStruct((NPAD, D), jnp.float32),
    )(meta, xd, w1, w2)


def _combine_body(y_hbm, pos1_hbm, pos2_hbm, g1_hbm, g2_hbm, out_hbm,
                  p1v, p2v, g1v, g2v, bufa, bufb, gsem, wsem):
    wid = lax.axis_index("s") * 2 + lax.axis_index("c")
    tb = wid * TPW
    pltpu.sync_copy(pos1_hbm.at[pl.ds(tb, TPW)], p1v)
    pltpu.sync_copy(pos2_hbm.at[pl.ds(tb, TPW)], p2v)
    pltpu.sync_copy(g1_hbm.at[pl.ds(tb, TPW)], g1v)
    pltpu.sync_copy(g2_hbm.at[pl.ds(tb, TPW)], g2v)
    nch = TPW // C2

    def issue(c):
        off = c * C2
        slot = c % NSLOT
        d1 = pltpu.async_copy(y_hbm.at[p1v.at[pl.ds(off, C2)]],
                              bufa.at[slot], gsem)
        d2 = pltpu.async_copy(y_hbm.at[p2v.at[pl.ds(off, C2)]],
                              bufb.at[slot], gsem)
        return d1, d2

    gds = [None] * nch
    wds = [None] * nch
    for c in range(NSLOT):
        gds[c] = issue(c)
    for c in range(nch):
        slot = c % NSLOT
        off = c * C2
        gds[c][0].wait()
        gds[c][1].wait()

        def rbody(r, _):
            i16 = jnp.broadcast_to(off + r, (16,))
            ga = plsc.load_gather(g1v, [i16])
            gb = plsc.load_gather(g2v, [i16])
            for j in range(D // 16):
                sl = pl.ds(j * 16, 16)
                bufa[slot, r, sl] = ga * bufa[slot, r, sl] + gb * bufb[slot, r, sl]
            return 0
        lax.fori_loop(0, C2, rbody, 0)
        wds[c] = pltpu.async_copy(bufa.at[slot],
                                  out_hbm.at[pl.ds(tb + off, C2)], wsem)
        if c + NSLOT < nch:
            wds[c].wait()
            gds[c + NSLOT] = issue(c + NSLOT)
    for c in range(max(0, nch - NSLOT), nch):
        wds[c].wait()


def _combine(y, pos1f, pos2f, g1f, g2f):
    fn = functools.partial(
        pl.kernel,
        out_type=jax.ShapeDtypeStruct((T, D), jnp.float32),
        mesh=plsc.VectorSubcoreMesh(core_axis_name="c", subcore_axis_name="s"),
        compiler_params=pltpu.CompilerParams(needs_layout_passes=False),
        scratch_types=[
            pltpu.VMEM((TPW,), jnp.int32),
            pltpu.VMEM((TPW,), jnp.int32),
            pltpu.VMEM((TPW,), jnp.float32),
            pltpu.VMEM((TPW,), jnp.float32),
            pltpu.VMEM((NSLOT, C2, D), jnp.float32),
            pltpu.VMEM((NSLOT, C2, D), jnp.float32),
            pltpu.SemaphoreType.DMA,
            pltpu.SemaphoreType.DMA,
        ],
    )(_combine_body)
    return fn(y, pos1f, pos2f, g1f, g2f)


@jax.jit
def kernel(inputs, gate_w, w1, w2):
    pos1, pos2, g1, g2, be = _routing(inputs, gate_w)
    pos1f = pos1.reshape(T)
    pos2f = pos2.reshape(T)
    xd = _dispatch(pos1f, pos2f, inputs)
    ucnt = be.reshape(128)[:NB]
    be_s = jnp.minimum(ucnt, E - 1)
    chg = jnp.concatenate(
        [jnp.ones((1,), jnp.int32), (be_s[1:] != be_s[:-1]).astype(jnp.int32)])
    run_id = jnp.cumsum(chg) - 1
    slot = run_id % 2
    re = jnp.zeros((NB,), jnp.int32).at[run_id].set(be_s)
    nxt = re[jnp.minimum(run_id + 1, run_id[-1])]
    meta = jnp.stack([be_s, chg, slot, nxt], axis=0)
    y = _ffn(meta, xd, w1, w2)
    return _combine(y, pos1f, pos2f, g1.reshape(T), g2.reshape(T))
